# Initial kernel scaffold; baseline (speedup 1.0000x reference)
import jax
import jax.numpy as jnp
from jax.experimental import pallas as pl


def kernel(initial_latents_nchw, text_embeddings, pooled_prompt_embeds, add_time_ids, w1, b1, wc, w2, b2, wt, wp, wid, bc, sigmas, timesteps):
    raise NotImplementedError("write your pallas kernel here")


# speedup vs baseline: 1.1305x; 1.1305x over previous
import numpy as np
import jax
import jax.numpy as jnp
from jax.experimental import pallas as pl
from jax.experimental.pallas import tpu as pltpu


def _make_denoise_kernel(H, W, hidden, cp, nsteps):
    hw = H * W
    offs = [(dy - 1) * W + (dx - 1) for dy in range(3) for dx in range(3)]

    def _body(scal_ref, x0_ref, add_ref, mask_ref, w1_ref, w2_ref, b2_ref,
              out_ref):
        m9 = mask_ref[...]
        m9b = m9.astype(jnp.bfloat16)
        w1 = w1_ref[...]
        w2 = w2_ref[...]
        b2 = b2_ref[...]

        lat0 = x0_ref[0]

        def step(i, lat):
            inv_scale = scal_ref[i, 0]
            dt = scal_ref[i, 1]

            x = lat * inv_scale

            taps = []
            for k in range(9):
                if k == 4:
                    taps.append(x)
                else:
                    taps.append(pltpu.roll(x, shift=(-offs[k]) % hw, axis=1))
            x9 = jnp.concatenate(taps, axis=0).astype(jnp.bfloat16) * m9b

            z1 = jnp.dot(w1, x9,
                         preferred_element_type=jnp.float32)

            h = z1 + add_ref[0, i]
            h = h * jax.nn.sigmoid(h)

            z2 = jnp.dot(w2, h.astype(jnp.bfloat16),
                         preferred_element_type=jnp.float32)
            eps = z2[4 * cp:5 * cp, :]
            for k in range(9):
                if k == 4:
                    continue
                zk = pltpu.roll(z2[k * cp:(k + 1) * cp, :],
                                shift=(-offs[k]) % hw, axis=1)
                eps = eps + zk * m9[k * cp:(k + 1) * cp, :]
            eps = eps + b2

            return lat + eps * dt

        out_ref[0] = jax.lax.fori_loop(0, nsteps, step, lat0, unroll=True)

    return _body


def _boundary_masks(H, W):
    m = np.zeros((9, H, W), np.float32)
    for dy in range(3):
        for dx in range(3):
            k = dy * 3 + dx
            ys = slice(max(0, 1 - dy), H - max(0, dy - 1))
            xs = slice(max(0, 1 - dx), W - max(0, dx - 1))
            m[k, ys, xs] = 1.0
    return jnp.asarray(m.reshape(9, H * W))


def _timestep_embedding(t, dim):
    half = dim // 2
    freqs = jnp.exp(-jnp.log(10000.0) * jnp.arange(half, dtype=jnp.float32) / half)
    args = t.astype(jnp.float32)[:, None] * freqs[None, :]
    return jnp.concatenate([jnp.cos(args), jnp.sin(args)], axis=-1)


def kernel(initial_latents_nchw, text_embeddings, pooled_prompt_embeds,
           add_time_ids, w1, b1, wc, w2, b2, wt, wp, wid, bc,
           sigmas, timesteps):
    B, C, H, W = initial_latents_nchw.shape
    HW = H * W
    nsteps = timesteps.shape[0]
    hidden = w1.shape[-1]
    CP = -(-C // 8) * 8

    sig = sigmas.astype(jnp.float32)

    x0 = initial_latents_nchw.astype(jnp.float32).reshape(B, C, HW)
    x0 = jnp.pad(x0, ((0, 0), (0, CP - C), (0, 0))) * jnp.sqrt(sig[0] ** 2 + 1.0)

    temb = _timestep_embedding(timesteps, wt.shape[0])
    t_part = temb @ wt
    p_part = pooled_prompt_embeds.astype(jnp.float32) @ wp
    i_part = add_time_ids.astype(jnp.float32) @ wid
    cond = jax.nn.silu(t_part[:, None, :] + (p_part + i_part)[None, :, :]
                       + bc[None, None, :])
    ctx = jnp.mean(text_embeddings.astype(jnp.float32), axis=1) @ wc
    add = cond + ctx[None] + b1.reshape(1, 1, hidden)
    addv = jnp.transpose(add, (1, 0, 2))[:, :, :, None]

    w1p = jnp.pad(w1, ((0, 0), (0, CP - C), (0, 0)))
    w1_i2c = jnp.transpose(w1p, (2, 0, 1)).reshape(hidden, 9 * CP)
    w1_i2c = w1_i2c.astype(jnp.bfloat16)
    w2p = jnp.pad(w2, ((0, 0), (0, 0), (0, CP - C)))
    w2cat = jnp.transpose(w2p, (0, 2, 1)).reshape(9 * CP, hidden)
    w2cat = w2cat.astype(jnp.bfloat16)
    b2v = jnp.pad(b2.reshape(-1), (0, CP - C)).reshape(CP, 1)

    m = _boundary_masks(H, W)
    mask9 = jnp.broadcast_to(m[:, None, :], (9, CP, HW)).reshape(9 * CP, HW)

    scal = jnp.stack([1.0 / jnp.sqrt(sig[:-1] ** 2 + 1.0),
                      sig[1:] - sig[:-1]], axis=1)

    out = pl.pallas_call(
        _make_denoise_kernel(H, W, hidden, CP, nsteps),
        out_shape=jax.ShapeDtypeStruct((B, CP, HW), jnp.float32),
        grid=(B,),
        in_specs=[
            pl.BlockSpec(memory_space=pltpu.MemorySpace.SMEM),
            pl.BlockSpec((1, CP, HW), lambda b: (b, 0, 0)),
            pl.BlockSpec((1, nsteps, hidden, 1), lambda b: (b, 0, 0, 0)),
            pl.BlockSpec((9 * CP, HW), lambda b: (0, 0)),
            pl.BlockSpec((hidden, 9 * CP), lambda b: (0, 0)),
            pl.BlockSpec((9 * CP, hidden), lambda b: (0, 0)),
            pl.BlockSpec((CP, 1), lambda b: (0, 0)),
        ],
        out_specs=pl.BlockSpec((1, CP, HW), lambda b: (b, 0, 0)),
        compiler_params=pltpu.CompilerParams(
            dimension_semantics=("parallel",)),
    )(scal, x0, addv, mask9, w1_i2c, w2cat, b2v)

    return out[:, :C, :].reshape(B, C, H, W)


# speedup vs baseline: 1.3845x; 1.2246x over previous
import numpy as np
import jax
import jax.numpy as jnp
from jax.experimental import pallas as pl
from jax.experimental.pallas import tpu as pltpu


def _make_denoise_kernel(H, W, hidden, cp, nsteps):
    hw = H * W
    offs = [(dy - 1) * W + (dx - 1) for dy in range(3) for dx in range(3)]

    def _body(scal_ref, x0_ref, add_ref, mask_ref, w1_ref, w2_ref, b2_ref,
              out_ref):
        m9 = mask_ref[...]
        m9b = m9.astype(jnp.bfloat16)
        w1 = w1_ref[...]
        w2 = w2_ref[...]
        b2 = b2_ref[...]

        lat0 = x0_ref[0]

        def step(i, lat):
            inv_scale = scal_ref[i, 0]
            dt = scal_ref[i, 1]

            x = lat * inv_scale

            taps = []
            for k in range(9):
                if k == 4:
                    taps.append(x.astype(jnp.bfloat16))
                else:
                    taps.append(
                        pltpu.roll(x, shift=(-offs[k]) % hw,
                                   axis=1).astype(jnp.bfloat16))
            x9 = jnp.concatenate(taps, axis=0) * m9b

            z1 = jnp.dot(w1, x9,
                         preferred_element_type=jnp.float32
                         ).astype(jnp.bfloat16)

            h = z1 + add_ref[0, i]
            h = h * (0.5 + 0.5 * jnp.tanh(0.5 * h))

            z2 = jnp.dot(w2, h,
                         preferred_element_type=jnp.float32)
            eps = z2[4 * cp:5 * cp, :]
            for k in range(9):
                if k == 4:
                    continue
                zk = pltpu.roll(z2[k * cp:(k + 1) * cp, :],
                                shift=(-offs[k]) % hw, axis=1)
                eps = eps + zk * m9[k * cp:(k + 1) * cp, :]
            eps = eps + b2

            return lat + eps * dt

        out_ref[0] = jax.lax.fori_loop(0, nsteps, step, lat0, unroll=True)

    return _body


def _boundary_masks(H, W):
    m = np.zeros((9, H, W), np.float32)
    for dy in range(3):
        for dx in range(3):
            k = dy * 3 + dx
            ys = slice(max(0, 1 - dy), H - max(0, dy - 1))
            xs = slice(max(0, 1 - dx), W - max(0, dx - 1))
            m[k, ys, xs] = 1.0
    return jnp.asarray(m.reshape(9, H * W))


def _timestep_embedding(t, dim):
    half = dim // 2
    freqs = jnp.exp(-jnp.log(10000.0) * jnp.arange(half, dtype=jnp.float32) / half)
    args = t.astype(jnp.float32)[:, None] * freqs[None, :]
    return jnp.concatenate([jnp.cos(args), jnp.sin(args)], axis=-1)


def kernel(initial_latents_nchw, text_embeddings, pooled_prompt_embeds,
           add_time_ids, w1, b1, wc, w2, b2, wt, wp, wid, bc,
           sigmas, timesteps):
    B, C, H, W = initial_latents_nchw.shape
    HW = H * W
    nsteps = timesteps.shape[0]
    hidden = w1.shape[-1]
    CP = -(-C // 8) * 8

    sig = sigmas.astype(jnp.float32)

    x0 = initial_latents_nchw.astype(jnp.float32).reshape(B, C, HW)
    x0 = jnp.pad(x0, ((0, 0), (0, CP - C), (0, 0))) * jnp.sqrt(sig[0] ** 2 + 1.0)

    temb = _timestep_embedding(timesteps, wt.shape[0])
    t_part = temb @ wt
    p_part = pooled_prompt_embeds.astype(jnp.float32) @ wp
    i_part = add_time_ids.astype(jnp.float32) @ wid
    cond = jax.nn.silu(t_part[:, None, :] + (p_part + i_part)[None, :, :]
                       + bc[None, None, :])
    ctx = jnp.mean(text_embeddings.astype(jnp.float32), axis=1) @ wc
    add = cond + ctx[None] + b1.reshape(1, 1, hidden)
    addv = jnp.transpose(add, (1, 0, 2))[:, :, :, None]
    addv = addv.astype(jnp.bfloat16)

    w1p = jnp.pad(w1, ((0, 0), (0, CP - C), (0, 0)))
    w1_i2c = jnp.transpose(w1p, (2, 0, 1)).reshape(hidden, 9 * CP)
    w1_i2c = w1_i2c.astype(jnp.bfloat16)
    w2p = jnp.pad(w2, ((0, 0), (0, 0), (0, CP - C)))
    w2cat = jnp.transpose(w2p, (0, 2, 1)).reshape(9 * CP, hidden)
    w2cat = w2cat.astype(jnp.bfloat16)
    b2v = jnp.pad(b2.reshape(-1), (0, CP - C)).reshape(CP, 1)

    m = _boundary_masks(H, W)
    mask9 = jnp.broadcast_to(m[:, None, :], (9, CP, HW)).reshape(9 * CP, HW)

    scal = jnp.stack([1.0 / jnp.sqrt(sig[:-1] ** 2 + 1.0),
                      sig[1:] - sig[:-1]], axis=1)

    out = pl.pallas_call(
        _make_denoise_kernel(H, W, hidden, CP, nsteps),
        out_shape=jax.ShapeDtypeStruct((B, CP, HW), jnp.float32),
        grid=(B,),
        in_specs=[
            pl.BlockSpec(memory_space=pltpu.MemorySpace.SMEM),
            pl.BlockSpec((1, CP, HW), lambda b: (b, 0, 0)),
            pl.BlockSpec((1, nsteps, hidden, 1), lambda b: (b, 0, 0, 0)),
            pl.BlockSpec((9 * CP, HW), lambda b: (0, 0)),
            pl.BlockSpec((hidden, 9 * CP), lambda b: (0, 0)),
            pl.BlockSpec((9 * CP, hidden), lambda b: (0, 0)),
            pl.BlockSpec((CP, 1), lambda b: (0, 0)),
        ],
        out_specs=pl.BlockSpec((1, CP, HW), lambda b: (b, 0, 0)),
        compiler_params=pltpu.CompilerParams(
            dimension_semantics=("parallel",)),
    )(scal, x0, addv, mask9, w1_i2c, w2cat, b2v)

    return out[:, :C, :].reshape(B, C, H, W)


# speedup vs baseline: 1.8510x; 1.3370x over previous
import numpy as np
import jax
import jax.numpy as jnp
from jax.experimental import pallas as pl
from jax.experimental.pallas import tpu as pltpu


def _make_denoise_kernel(H, W, hidden, cp, nsteps):
    hw = H * W
    offs = [(dy - 1) * W + (dx - 1) for dy in range(3) for dx in range(3)]

    def _body(scal_ref, x0_ref, add_ref, mask_ref, w1_ref, w2_ref, b2_ref,
              out_ref):
        m9 = mask_ref[...]
        m9b = m9.astype(jnp.bfloat16)
        w1 = w1_ref[...]
        w2 = w2_ref[...]
        b2 = b2_ref[...]

        def one(lat, inv_scale, dt, a_col):
            x = lat * inv_scale

            taps = []
            for k in range(9):
                if k == 4:
                    taps.append(x)
                else:
                    taps.append(pltpu.roll(x, shift=(-offs[k]) % hw, axis=1))
            x9 = jnp.concatenate(taps, axis=0).astype(jnp.bfloat16) * m9b

            z1 = jnp.dot(w1, x9,
                         preferred_element_type=jnp.float32
                         ).astype(jnp.bfloat16)

            h = z1 + a_col
            h = h * (0.5 + 0.5 * jnp.tanh(0.5 * h))

            z2 = jnp.dot(w2, h,
                         preferred_element_type=jnp.float32)
            eps = z2[4 * cp:5 * cp, :]
            for k in range(9):
                if k == 4:
                    continue
                zk = pltpu.roll(z2[k * cp:(k + 1) * cp, :],
                                shift=(-offs[k]) % hw, axis=1)
                eps = eps + zk * m9[k * cp:(k + 1) * cp, :]
            eps = eps + b2

            return lat + eps * dt

        def step(i, lats):
            inv_scale = scal_ref[i, 0]
            dt = scal_ref[i, 1]
            la, lb = lats
            la = one(la, inv_scale, dt, add_ref[0, i])
            lb = one(lb, inv_scale, dt, add_ref[1, i])
            return (la, lb)

        outs = jax.lax.fori_loop(0, nsteps, step, (x0_ref[0], x0_ref[1]),
                                 unroll=True)
        out_ref[0] = outs[0]
        out_ref[1] = outs[1]

    return _body


def _boundary_masks(H, W):
    m = np.zeros((9, H, W), np.float32)
    for dy in range(3):
        for dx in range(3):
            k = dy * 3 + dx
            ys = slice(max(0, 1 - dy), H - max(0, dy - 1))
            xs = slice(max(0, 1 - dx), W - max(0, dx - 1))
            m[k, ys, xs] = 1.0
    return jnp.asarray(m.reshape(9, H * W))


def _timestep_embedding(t, dim):
    half = dim // 2
    freqs = jnp.exp(-jnp.log(10000.0) * jnp.arange(half, dtype=jnp.float32) / half)
    args = t.astype(jnp.float32)[:, None] * freqs[None, :]
    return jnp.concatenate([jnp.cos(args), jnp.sin(args)], axis=-1)


def kernel(initial_latents_nchw, text_embeddings, pooled_prompt_embeds,
           add_time_ids, w1, b1, wc, w2, b2, wt, wp, wid, bc,
           sigmas, timesteps):
    B, C, H, W = initial_latents_nchw.shape
    HW = H * W
    nsteps = timesteps.shape[0]
    hidden = w1.shape[-1]
    CP = -(-C // 8) * 8

    sig = sigmas.astype(jnp.float32)

    x0 = initial_latents_nchw.astype(jnp.float32).reshape(B, C, HW)
    x0 = jnp.pad(x0, ((0, 0), (0, CP - C), (0, 0))) * jnp.sqrt(sig[0] ** 2 + 1.0)

    temb = _timestep_embedding(timesteps, wt.shape[0])
    t_part = temb @ wt
    p_part = pooled_prompt_embeds.astype(jnp.float32) @ wp
    i_part = add_time_ids.astype(jnp.float32) @ wid
    cond = jax.nn.silu(t_part[:, None, :] + (p_part + i_part)[None, :, :]
                       + bc[None, None, :])
    ctx = jnp.mean(text_embeddings.astype(jnp.float32), axis=1) @ wc
    add = cond + ctx[None] + b1.reshape(1, 1, hidden)
    addv = jnp.transpose(add, (1, 0, 2))[:, :, :, None]
    addv = addv.astype(jnp.bfloat16)

    w1p = jnp.pad(w1, ((0, 0), (0, CP - C), (0, 0)))
    w1_i2c = jnp.transpose(w1p, (2, 0, 1)).reshape(hidden, 9 * CP)
    w1_i2c = w1_i2c.astype(jnp.bfloat16)
    w2p = jnp.pad(w2, ((0, 0), (0, 0), (0, CP - C)))
    w2cat = jnp.transpose(w2p, (0, 2, 1)).reshape(9 * CP, hidden)
    w2cat = w2cat.astype(jnp.bfloat16)
    b2v = jnp.pad(b2.reshape(-1), (0, CP - C)).reshape(CP, 1)

    m = _boundary_masks(H, W)
    mask9 = jnp.broadcast_to(m[:, None, :], (9, CP, HW)).reshape(9 * CP, HW)

    scal = jnp.stack([1.0 / jnp.sqrt(sig[:-1] ** 2 + 1.0),
                      sig[1:] - sig[:-1]], axis=1)

    out = pl.pallas_call(
        _make_denoise_kernel(H, W, hidden, CP, nsteps),
        out_shape=jax.ShapeDtypeStruct((B, CP, HW), jnp.float32),
        grid=(B // 2,),
        in_specs=[
            pl.BlockSpec(memory_space=pltpu.MemorySpace.SMEM),
            pl.BlockSpec((2, CP, HW), lambda b: (b, 0, 0)),
            pl.BlockSpec((2, nsteps, hidden, 1), lambda b: (b, 0, 0, 0)),
            pl.BlockSpec((9 * CP, HW), lambda b: (0, 0)),
            pl.BlockSpec((hidden, 9 * CP), lambda b: (0, 0)),
            pl.BlockSpec((9 * CP, hidden), lambda b: (0, 0)),
            pl.BlockSpec((CP, 1), lambda b: (0, 0)),
        ],
        out_specs=pl.BlockSpec((2, CP, HW), lambda b: (b, 0, 0)),
        compiler_params=pltpu.CompilerParams(
            dimension_semantics=("parallel",)),
    )(scal, x0, addv, mask9, w1_i2c, w2cat, b2v)

    return out[:, :C, :].reshape(B, C, H, W)


# speedup vs baseline: 1.8545x; 1.0019x over previous
import numpy as np
import jax
import jax.numpy as jnp
from jax.experimental import pallas as pl
from jax.experimental.pallas import tpu as pltpu


def _make_denoise_kernel(H, W, hidden, cp, nsteps, nchain):
    hw = H * W
    offs = [(dy - 1) * W + (dx - 1) for dy in range(3) for dx in range(3)]

    def _body(scal_ref, x0_ref, add_ref, mask_ref, w1_ref, w2_ref, b2_ref,
              out_ref):
        m9 = mask_ref[...]
        m9b = m9.astype(jnp.bfloat16)
        w1 = w1_ref[...]
        w2 = w2_ref[...]
        b2 = b2_ref[...]

        def one(lat, inv_scale, dt, a_col):
            x = lat * inv_scale

            taps = []
            for k in range(9):
                if k == 4:
                    taps.append(x)
                else:
                    taps.append(pltpu.roll(x, shift=(-offs[k]) % hw, axis=1))
            x9 = jnp.concatenate(taps, axis=0).astype(jnp.bfloat16) * m9b

            z1 = jnp.dot(w1, x9,
                         preferred_element_type=jnp.float32
                         ).astype(jnp.bfloat16)

            h = z1 + a_col
            h = h * (0.5 + 0.5 * jnp.tanh(0.5 * h))

            z2 = jnp.dot(w2, h,
                         preferred_element_type=jnp.float32)
            eps = z2[4 * cp:5 * cp, :]
            for k in range(9):
                if k == 4:
                    continue
                zk = pltpu.roll(z2[k * cp:(k + 1) * cp, :],
                                shift=(-offs[k]) % hw, axis=1)
                eps = eps + zk * m9[k * cp:(k + 1) * cp, :]
            eps = eps + b2

            return lat + eps * dt

        def step(i, lats):
            inv_scale = scal_ref[i, 0]
            dt = scal_ref[i, 1]
            return tuple(one(lats[j], inv_scale, dt, add_ref[j, i])
                         for j in range(nchain))

        outs = jax.lax.fori_loop(0, nsteps, step,
                                 tuple(x0_ref[j] for j in range(nchain)))
        for j in range(nchain):
            out_ref[j] = outs[j]

    return _body


def _boundary_masks(H, W):
    m = np.zeros((9, H, W), np.float32)
    for dy in range(3):
        for dx in range(3):
            k = dy * 3 + dx
            ys = slice(max(0, 1 - dy), H - max(0, dy - 1))
            xs = slice(max(0, 1 - dx), W - max(0, dx - 1))
            m[k, ys, xs] = 1.0
    return jnp.asarray(m.reshape(9, H * W))


def _timestep_embedding(t, dim):
    half = dim // 2
    freqs = jnp.exp(-jnp.log(10000.0) * jnp.arange(half, dtype=jnp.float32) / half)
    args = t.astype(jnp.float32)[:, None] * freqs[None, :]
    return jnp.concatenate([jnp.cos(args), jnp.sin(args)], axis=-1)


def kernel(initial_latents_nchw, text_embeddings, pooled_prompt_embeds,
           add_time_ids, w1, b1, wc, w2, b2, wt, wp, wid, bc,
           sigmas, timesteps):
    B, C, H, W = initial_latents_nchw.shape
    HW = H * W
    nsteps = timesteps.shape[0]
    hidden = w1.shape[-1]
    CP = -(-C // 8) * 8

    sig = sigmas.astype(jnp.float32)

    x0 = initial_latents_nchw.astype(jnp.float32).reshape(B, C, HW)
    x0 = jnp.pad(x0, ((0, 0), (0, CP - C), (0, 0))) * jnp.sqrt(sig[0] ** 2 + 1.0)

    temb = _timestep_embedding(timesteps, wt.shape[0])
    t_part = temb @ wt
    p_part = pooled_prompt_embeds.astype(jnp.float32) @ wp
    i_part = add_time_ids.astype(jnp.float32) @ wid
    cond = jax.nn.silu(t_part[:, None, :] + (p_part + i_part)[None, :, :]
                       + bc[None, None, :])
    ctx = jnp.mean(text_embeddings.astype(jnp.float32), axis=1) @ wc
    add = cond + ctx[None] + b1.reshape(1, 1, hidden)
    addv = jnp.transpose(add, (1, 0, 2))[:, :, :, None]
    addv = addv.astype(jnp.bfloat16)

    w1p = jnp.pad(w1, ((0, 0), (0, CP - C), (0, 0)))
    w1_i2c = jnp.transpose(w1p, (2, 0, 1)).reshape(hidden, 9 * CP)
    w1_i2c = w1_i2c.astype(jnp.bfloat16)
    w2p = jnp.pad(w2, ((0, 0), (0, 0), (0, CP - C)))
    w2cat = jnp.transpose(w2p, (0, 2, 1)).reshape(9 * CP, hidden)
    w2cat = w2cat.astype(jnp.bfloat16)
    b2v = jnp.pad(b2.reshape(-1), (0, CP - C)).reshape(CP, 1)

    m = _boundary_masks(H, W)
    mask9 = jnp.broadcast_to(m[:, None, :], (9, CP, HW)).reshape(9 * CP, HW)

    scal = jnp.stack([1.0 / jnp.sqrt(sig[:-1] ** 2 + 1.0),
                      sig[1:] - sig[:-1]], axis=1)

    NCHAIN = 4
    out = pl.pallas_call(
        _make_denoise_kernel(H, W, hidden, CP, nsteps, NCHAIN),
        out_shape=jax.ShapeDtypeStruct((B, CP, HW), jnp.float32),
        grid=(B // NCHAIN,),
        in_specs=[
            pl.BlockSpec(memory_space=pltpu.MemorySpace.SMEM),
            pl.BlockSpec((NCHAIN, CP, HW), lambda b: (b, 0, 0)),
            pl.BlockSpec((NCHAIN, nsteps, hidden, 1),
                         lambda b: (b, 0, 0, 0)),
            pl.BlockSpec((9 * CP, HW), lambda b: (0, 0)),
            pl.BlockSpec((hidden, 9 * CP), lambda b: (0, 0)),
            pl.BlockSpec((9 * CP, hidden), lambda b: (0, 0)),
            pl.BlockSpec((CP, 1), lambda b: (0, 0)),
        ],
        out_specs=pl.BlockSpec((NCHAIN, CP, HW), lambda b: (b, 0, 0)),
        compiler_params=pltpu.CompilerParams(
            dimension_semantics=("parallel",)),
    )(scal, x0, addv, mask9, w1_i2c, w2cat, b2v)

    return out[:, :C, :].reshape(B, C, H, W)


# speedup vs baseline: 1.8552x; 1.0004x over previous
import numpy as np
import jax
import jax.numpy as jnp
from jax.experimental import pallas as pl
from jax.experimental.pallas import tpu as pltpu


def _make_denoise_kernel(H, W, hidden, cp, nsteps, nchain):
    hw = H * W
    offs = [(dy - 1) * W + (dx - 1) for dy in range(3) for dx in range(3)]

    def _body(scal_ref, x0_ref, add_ref, mask_ref, w1_ref, w2_ref, b2_ref,
              out_ref):
        m9 = mask_ref[...]
        m9b = m9.astype(jnp.bfloat16)
        w1 = w1_ref[...]
        w2 = w2_ref[...]
        b2 = b2_ref[...]

        def one(lat, inv_scale, dt, a_col):
            x = lat * inv_scale

            taps = []
            for k in range(9):
                if k == 4:
                    taps.append(x)
                else:
                    taps.append(pltpu.roll(x, shift=(-offs[k]) % hw, axis=1))
            x9 = jnp.concatenate(taps, axis=0).astype(jnp.bfloat16) * m9b

            z1 = jnp.dot(w1, x9,
                         preferred_element_type=jnp.float32
                         ).astype(jnp.bfloat16)

            h = z1 + a_col
            h = h * (0.5 + 0.5 * jnp.tanh(0.5 * h))

            z2 = jnp.dot(w2, h,
                         preferred_element_type=jnp.float32)
            eps = z2[4 * cp:5 * cp, :]
            for k in range(9):
                if k == 4:
                    continue
                zk = pltpu.roll(z2[k * cp:(k + 1) * cp, :],
                                shift=(-offs[k]) % hw, axis=1)
                eps = eps + zk * m9[k * cp:(k + 1) * cp, :]
            eps = eps + b2

            return lat + eps * dt

        def step(i, lats):
            inv_scale = scal_ref[i, 0]
            dt = scal_ref[i, 1]
            return tuple(one(lats[j], inv_scale, dt, add_ref[j, i])
                         for j in range(nchain))

        outs = jax.lax.fori_loop(0, nsteps, step,
                                 tuple(x0_ref[j] for j in range(nchain)),
                                 unroll=2)
        for j in range(nchain):
            out_ref[j] = outs[j]

    return _body


def _boundary_masks(H, W):
    m = np.zeros((9, H, W), np.float32)
    for dy in range(3):
        for dx in range(3):
            k = dy * 3 + dx
            ys = slice(max(0, 1 - dy), H - max(0, dy - 1))
            xs = slice(max(0, 1 - dx), W - max(0, dx - 1))
            m[k, ys, xs] = 1.0
    return jnp.asarray(m.reshape(9, H * W))


def _timestep_embedding(t, dim):
    half = dim // 2
    freqs = jnp.exp(-jnp.log(10000.0) * jnp.arange(half, dtype=jnp.float32) / half)
    args = t.astype(jnp.float32)[:, None] * freqs[None, :]
    return jnp.concatenate([jnp.cos(args), jnp.sin(args)], axis=-1)


def kernel(initial_latents_nchw, text_embeddings, pooled_prompt_embeds,
           add_time_ids, w1, b1, wc, w2, b2, wt, wp, wid, bc,
           sigmas, timesteps):
    B, C, H, W = initial_latents_nchw.shape
    HW = H * W
    nsteps = timesteps.shape[0]
    hidden = w1.shape[-1]
    CP = -(-C // 8) * 8

    sig = sigmas.astype(jnp.float32)

    x0 = initial_latents_nchw.astype(jnp.float32).reshape(B, C, HW)
    x0 = jnp.pad(x0, ((0, 0), (0, CP - C), (0, 0))) * jnp.sqrt(sig[0] ** 2 + 1.0)

    temb = _timestep_embedding(timesteps, wt.shape[0])
    t_part = temb @ wt
    p_part = pooled_prompt_embeds.astype(jnp.float32) @ wp
    i_part = add_time_ids.astype(jnp.float32) @ wid
    cond = jax.nn.silu(t_part[:, None, :] + (p_part + i_part)[None, :, :]
                       + bc[None, None, :])
    ctx = jnp.mean(text_embeddings.astype(jnp.float32), axis=1) @ wc
    add = cond + ctx[None] + b1.reshape(1, 1, hidden)
    addv = jnp.transpose(add, (1, 0, 2))[:, :, :, None]
    addv = addv.astype(jnp.bfloat16)

    w1p = jnp.pad(w1, ((0, 0), (0, CP - C), (0, 0)))
    w1_i2c = jnp.transpose(w1p, (2, 0, 1)).reshape(hidden, 9 * CP)
    w1_i2c = w1_i2c.astype(jnp.bfloat16)
    w2p = jnp.pad(w2, ((0, 0), (0, 0), (0, CP - C)))
    w2cat = jnp.transpose(w2p, (0, 2, 1)).reshape(9 * CP, hidden)
    w2cat = w2cat.astype(jnp.bfloat16)
    b2v = jnp.pad(b2.reshape(-1), (0, CP - C)).reshape(CP, 1)

    m = _boundary_masks(H, W)
    mask9 = jnp.broadcast_to(m[:, None, :], (9, CP, HW)).reshape(9 * CP, HW)

    scal = jnp.stack([1.0 / jnp.sqrt(sig[:-1] ** 2 + 1.0),
                      sig[1:] - sig[:-1]], axis=1)

    NCHAIN = max(d for d in (4, 2, 1) if B % d == 0)
    out = pl.pallas_call(
        _make_denoise_kernel(H, W, hidden, CP, nsteps, NCHAIN),
        out_shape=jax.ShapeDtypeStruct((B, CP, HW), jnp.float32),
        grid=(B // NCHAIN,),
        in_specs=[
            pl.BlockSpec(memory_space=pltpu.MemorySpace.SMEM),
            pl.BlockSpec((NCHAIN, CP, HW), lambda b: (b, 0, 0)),
            pl.BlockSpec((NCHAIN, nsteps, hidden, 1),
                         lambda b: (b, 0, 0, 0)),
            pl.BlockSpec((9 * CP, HW), lambda b: (0, 0)),
            pl.BlockSpec((hidden, 9 * CP), lambda b: (0, 0)),
            pl.BlockSpec((9 * CP, hidden), lambda b: (0, 0)),
            pl.BlockSpec((CP, 1), lambda b: (0, 0)),
        ],
        out_specs=pl.BlockSpec((NCHAIN, CP, HW), lambda b: (b, 0, 0)),
        compiler_params=pltpu.CompilerParams(
            dimension_semantics=("parallel",)),
    )(scal, x0, addv, mask9, w1_i2c, w2cat, b2v)

    return out[:, :C, :].reshape(B, C, H, W)


# speedup vs baseline: 1.8935x; 1.0206x over previous
import numpy as np
import jax
import jax.numpy as jnp
from jax.experimental import pallas as pl
from jax.experimental.pallas import tpu as pltpu


def _make_denoise_kernel(H, W, hidden, cp, nsteps, nchain):
    hw = H * W
    offs = [(dy - 1) * W + (dx - 1) for dy in range(3) for dx in range(3)]

    def _body(x0_ref, add_ref, mask_ref, w1_ref, w2_ref, b2_ref,
              out_ref):
        m9 = mask_ref[...]
        m9b = m9.astype(jnp.bfloat16)

        def one(lat, w1s, w2s, b2s, a_col):
            taps = []
            for k in range(9):
                if k == 4:
                    taps.append(lat)
                else:
                    taps.append(pltpu.roll(lat, shift=(-offs[k]) % hw, axis=1))
            x9 = jnp.concatenate(taps, axis=0).astype(jnp.bfloat16) * m9b

            z1 = jnp.dot(w1s, x9,
                         preferred_element_type=jnp.float32
                         ).astype(jnp.bfloat16)

            h = z1 + a_col
            h = h * (0.5 + 0.5 * jnp.tanh(0.5 * h))

            z2 = jnp.dot(w2s, h,
                         preferred_element_type=jnp.float32)
            eps = z2[4 * cp:5 * cp, :]
            for k in range(9):
                if k == 4:
                    continue
                zk = pltpu.roll(z2[k * cp:(k + 1) * cp, :],
                                shift=(-offs[k]) % hw, axis=1)
                eps = eps + zk * m9[k * cp:(k + 1) * cp, :]
            eps = eps + b2s

            return lat + eps

        def step(i, lats):
            w1s = w1_ref[i]
            w2s = w2_ref[i]
            b2s = b2_ref[i]
            return tuple(one(lats[j], w1s, w2s, b2s, add_ref[j, i])
                         for j in range(nchain))

        outs = jax.lax.fori_loop(0, nsteps, step,
                                 tuple(x0_ref[j] for j in range(nchain)),
                                 unroll=2)
        for j in range(nchain):
            out_ref[j] = outs[j]

    return _body


def _boundary_masks(H, W):
    m = np.zeros((9, H, W), np.float32)
    for dy in range(3):
        for dx in range(3):
            k = dy * 3 + dx
            ys = slice(max(0, 1 - dy), H - max(0, dy - 1))
            xs = slice(max(0, 1 - dx), W - max(0, dx - 1))
            m[k, ys, xs] = 1.0
    return jnp.asarray(m.reshape(9, H * W))


def _timestep_embedding(t, dim):
    half = dim // 2
    freqs = jnp.exp(-jnp.log(10000.0) * jnp.arange(half, dtype=jnp.float32) / half)
    args = t.astype(jnp.float32)[:, None] * freqs[None, :]
    return jnp.concatenate([jnp.cos(args), jnp.sin(args)], axis=-1)


def kernel(initial_latents_nchw, text_embeddings, pooled_prompt_embeds,
           add_time_ids, w1, b1, wc, w2, b2, wt, wp, wid, bc,
           sigmas, timesteps):
    B, C, H, W = initial_latents_nchw.shape
    HW = H * W
    nsteps = timesteps.shape[0]
    hidden = w1.shape[-1]
    CP = -(-C // 8) * 8

    sig = sigmas.astype(jnp.float32)

    x0 = initial_latents_nchw.astype(jnp.float32).reshape(B, C, HW)
    x0 = jnp.pad(x0, ((0, 0), (0, CP - C), (0, 0))) * jnp.sqrt(sig[0] ** 2 + 1.0)

    temb = _timestep_embedding(timesteps, wt.shape[0])
    t_part = temb @ wt
    p_part = pooled_prompt_embeds.astype(jnp.float32) @ wp
    i_part = add_time_ids.astype(jnp.float32) @ wid
    cond = jax.nn.silu(t_part[:, None, :] + (p_part + i_part)[None, :, :]
                       + bc[None, None, :])
    ctx = jnp.mean(text_embeddings.astype(jnp.float32), axis=1) @ wc
    add = cond + ctx[None] + b1.reshape(1, 1, hidden)
    addv = jnp.transpose(add, (1, 0, 2))[:, :, :, None]
    addv = addv.astype(jnp.bfloat16)

    inv_scale = 1.0 / jnp.sqrt(sig[:-1] ** 2 + 1.0)
    dtv = sig[1:] - sig[:-1]
    w1p = jnp.pad(w1, ((0, 0), (0, CP - C), (0, 0)))
    w1_i2c = jnp.transpose(w1p, (2, 0, 1)).reshape(hidden, 9 * CP)
    w1_steps = (w1_i2c[None] * inv_scale[:, None, None]).astype(jnp.bfloat16)
    w2p = jnp.pad(w2, ((0, 0), (0, 0), (0, CP - C)))
    w2cat = jnp.transpose(w2p, (0, 2, 1)).reshape(9 * CP, hidden)
    w2_steps = (w2cat[None] * dtv[:, None, None]).astype(jnp.bfloat16)
    b2v = jnp.pad(b2.reshape(-1), (0, CP - C)).reshape(1, CP, 1)
    b2_steps = b2v * dtv[:, None, None]

    m = _boundary_masks(H, W)
    mask9 = jnp.broadcast_to(m[:, None, :], (9, CP, HW)).reshape(9 * CP, HW)

    NCHAIN = max(d for d in (4, 2, 1) if B % d == 0)
    out = pl.pallas_call(
        _make_denoise_kernel(H, W, hidden, CP, nsteps, NCHAIN),
        out_shape=jax.ShapeDtypeStruct((B, CP, HW), jnp.float32),
        grid=(B // NCHAIN,),
        in_specs=[
            pl.BlockSpec((NCHAIN, CP, HW), lambda b: (b, 0, 0)),
            pl.BlockSpec((NCHAIN, nsteps, hidden, 1),
                         lambda b: (b, 0, 0, 0)),
            pl.BlockSpec((9 * CP, HW), lambda b: (0, 0)),
            pl.BlockSpec((nsteps, hidden, 9 * CP), lambda b: (0, 0, 0)),
            pl.BlockSpec((nsteps, 9 * CP, hidden), lambda b: (0, 0, 0)),
            pl.BlockSpec((nsteps, CP, 1), lambda b: (0, 0, 0)),
        ],
        out_specs=pl.BlockSpec((NCHAIN, CP, HW), lambda b: (b, 0, 0)),
        compiler_params=pltpu.CompilerParams(
            dimension_semantics=("parallel",)),
    )(x0, addv, mask9, w1_steps, w2_steps, b2_steps)

    return out[:, :C, :].reshape(B, C, H, W)


# speedup vs baseline: 1.9008x; 1.0039x over previous
import numpy as np
import jax
import jax.numpy as jnp
from jax.experimental import pallas as pl
from jax.experimental.pallas import tpu as pltpu


def _make_denoise_kernel(H, W, hidden, cp, nsteps, nchain):
    hw = H * W
    offs = [(dy - 1) * W + (dx - 1) for dy in range(3) for dx in range(3)]

    def _body(x0_ref, add_ref, mask_ref, w1_ref, w2_ref, b2_ref,
              out_ref):
        m9 = mask_ref[...]
        m9b = m9.astype(jnp.bfloat16)

        def one(lat, w1s, w2s, b2s, a_col):
            taps = []
            for k in range(9):
                if k == 4:
                    taps.append(lat)
                else:
                    taps.append(pltpu.roll(lat, shift=(-offs[k]) % hw, axis=1))
            x9 = jnp.concatenate(taps, axis=0).astype(jnp.bfloat16) * m9b

            z1 = jnp.dot(w1s, x9,
                         preferred_element_type=jnp.float32
                         ).astype(jnp.bfloat16)

            h = z1 + a_col
            h = h * (0.5 + 0.5 * jnp.tanh(0.5 * h))

            z2 = jnp.dot(w2s, h,
                         preferred_element_type=jnp.float32)
            eps = z2[4 * cp:5 * cp, :]
            for k in range(9):
                if k == 4:
                    continue
                zk = pltpu.roll(z2[k * cp:(k + 1) * cp, :],
                                shift=(-offs[k]) % hw, axis=1)
                eps = eps + zk * m9[k * cp:(k + 1) * cp, :]
            eps = eps + b2s

            return lat + eps

        def step(i, lats):
            w1s = w1_ref[i]
            w2s = w2_ref[i]
            b2s = b2_ref[i]
            return tuple(one(lats[j], w1s, w2s, b2s, add_ref[j, i])
                         for j in range(nchain))

        outs = jax.lax.fori_loop(0, nsteps, step,
                                 tuple(x0_ref[j] for j in range(nchain)),
                                 unroll=4)
        for j in range(nchain):
            out_ref[j] = outs[j]

    return _body


def _boundary_masks(H, W):
    m = np.zeros((9, H, W), np.float32)
    for dy in range(3):
        for dx in range(3):
            k = dy * 3 + dx
            ys = slice(max(0, 1 - dy), H - max(0, dy - 1))
            xs = slice(max(0, 1 - dx), W - max(0, dx - 1))
            m[k, ys, xs] = 1.0
    return jnp.asarray(m.reshape(9, H * W))


def _timestep_embedding(t, dim):
    half = dim // 2
    freqs = jnp.exp(-jnp.log(10000.0) * jnp.arange(half, dtype=jnp.float32) / half)
    args = t.astype(jnp.float32)[:, None] * freqs[None, :]
    return jnp.concatenate([jnp.cos(args), jnp.sin(args)], axis=-1)


def kernel(initial_latents_nchw, text_embeddings, pooled_prompt_embeds,
           add_time_ids, w1, b1, wc, w2, b2, wt, wp, wid, bc,
           sigmas, timesteps):
    B, C, H, W = initial_latents_nchw.shape
    HW = H * W
    nsteps = timesteps.shape[0]
    hidden = w1.shape[-1]
    CP = -(-C // 8) * 8

    sig = sigmas.astype(jnp.float32)

    x0 = initial_latents_nchw.astype(jnp.float32).reshape(B, C, HW)
    x0 = jnp.pad(x0, ((0, 0), (0, CP - C), (0, 0))) * jnp.sqrt(sig[0] ** 2 + 1.0)

    temb = _timestep_embedding(timesteps, wt.shape[0])
    t_part = temb @ wt
    p_part = pooled_prompt_embeds.astype(jnp.float32) @ wp
    i_part = add_time_ids.astype(jnp.float32) @ wid
    cond = jax.nn.silu(t_part[:, None, :] + (p_part + i_part)[None, :, :]
                       + bc[None, None, :])
    ctx = jnp.mean(text_embeddings.astype(jnp.float32), axis=1) @ wc
    add = cond + ctx[None] + b1.reshape(1, 1, hidden)
    addv = jnp.transpose(add, (1, 0, 2))[:, :, :, None]
    addv = addv.astype(jnp.bfloat16)

    inv_scale = 1.0 / jnp.sqrt(sig[:-1] ** 2 + 1.0)
    dtv = sig[1:] - sig[:-1]
    w1p = jnp.pad(w1, ((0, 0), (0, CP - C), (0, 0)))
    w1_i2c = jnp.transpose(w1p, (2, 0, 1)).reshape(hidden, 9 * CP)
    w1_steps = (w1_i2c[None] * inv_scale[:, None, None]).astype(jnp.bfloat16)
    w2p = jnp.pad(w2, ((0, 0), (0, 0), (0, CP - C)))
    w2cat = jnp.transpose(w2p, (0, 2, 1)).reshape(9 * CP, hidden)
    w2_steps = (w2cat[None] * dtv[:, None, None]).astype(jnp.bfloat16)
    b2v = jnp.pad(b2.reshape(-1), (0, CP - C)).reshape(1, CP, 1)
    b2_steps = b2v * dtv[:, None, None]

    m = _boundary_masks(H, W)
    mask9 = jnp.broadcast_to(m[:, None, :], (9, CP, HW)).reshape(9 * CP, HW)

    NCHAIN = max(d for d in (4, 2, 1) if B % d == 0)
    out = pl.pallas_call(
        _make_denoise_kernel(H, W, hidden, CP, nsteps, NCHAIN),
        out_shape=jax.ShapeDtypeStruct((B, CP, HW), jnp.float32),
        grid=(B // NCHAIN,),
        in_specs=[
            pl.BlockSpec((NCHAIN, CP, HW), lambda b: (b, 0, 0)),
            pl.BlockSpec((NCHAIN, nsteps, hidden, 1),
                         lambda b: (b, 0, 0, 0)),
            pl.BlockSpec((9 * CP, HW), lambda b: (0, 0)),
            pl.BlockSpec((nsteps, hidden, 9 * CP), lambda b: (0, 0, 0)),
            pl.BlockSpec((nsteps, 9 * CP, hidden), lambda b: (0, 0, 0)),
            pl.BlockSpec((nsteps, CP, 1), lambda b: (0, 0, 0)),
        ],
        out_specs=pl.BlockSpec((NCHAIN, CP, HW), lambda b: (b, 0, 0)),
        compiler_params=pltpu.CompilerParams(
            dimension_semantics=("parallel",)),
    )(x0, addv, mask9, w1_steps, w2_steps, b2_steps)

    return out[:, :C, :].reshape(B, C, H, W)


# speedup vs baseline: 1.9649x; 1.0338x over previous
import numpy as np
import jax
import jax.numpy as jnp
from jax.experimental import pallas as pl
from jax.experimental.pallas import tpu as pltpu


def _make_denoise_kernel(H, W, hidden, cp, nsteps, nchain):
    hw = H * W
    offs = [(dy - 1) * W + (dx - 1) for dy in range(3) for dx in range(3)]

    def _body(x0_ref, add_ref, mask_ref, w1_ref, w2_ref, b2_ref,
              out_ref):
        m9 = mask_ref[...]
        m9b = m9.astype(jnp.bfloat16)

        def one(lat, w1s, w2s, b2s, a_col):
            taps = []
            for k in range(9):
                if k == 4:
                    taps.append(lat)
                else:
                    taps.append(pltpu.roll(lat, shift=(-offs[k]) % hw, axis=1))
            x9 = jnp.concatenate(taps, axis=0).astype(jnp.bfloat16) * m9b

            z1 = jnp.dot(w1s, x9,
                         preferred_element_type=jnp.float32
                         ).astype(jnp.bfloat16)

            h = z1 + a_col
            h = h * (0.5 + 0.5 * jnp.tanh(0.5 * h))

            z2 = jnp.dot(w2s, h,
                         preferred_element_type=jnp.float32)
            eps = z2[4 * cp:5 * cp, :]
            for k in range(9):
                if k == 4:
                    continue
                zk = pltpu.roll(z2[k * cp:(k + 1) * cp, :],
                                shift=(-offs[k]) % hw, axis=1)
                eps = eps + zk * m9[k * cp:(k + 1) * cp, :]
            eps = eps + b2s

            return lat + eps

        def step(i, lats):
            w1s = w1_ref[i]
            w2s = w2_ref[i]
            b2s = b2_ref[i]
            return tuple(one(lats[j], w1s, w2s, b2s, add_ref[j, i])
                         for j in range(nchain))

        outs = jax.lax.fori_loop(0, nsteps, step,
                                 tuple(x0_ref[j] for j in range(nchain)),
                                 unroll=2)
        for j in range(nchain):
            out_ref[j] = outs[j]

    return _body


def _boundary_masks(H, W):
    m = np.zeros((9, H, W), np.float32)
    for dy in range(3):
        for dx in range(3):
            k = dy * 3 + dx
            ys = slice(max(0, 1 - dy), H - max(0, dy - 1))
            xs = slice(max(0, 1 - dx), W - max(0, dx - 1))
            m[k, ys, xs] = 1.0
    return jnp.asarray(m.reshape(9, H * W))


def _timestep_embedding(t, dim):
    half = dim // 2
    freqs = jnp.exp(-jnp.log(10000.0) * jnp.arange(half, dtype=jnp.float32) / half)
    args = t.astype(jnp.float32)[:, None] * freqs[None, :]
    return jnp.concatenate([jnp.cos(args), jnp.sin(args)], axis=-1)


def kernel(initial_latents_nchw, text_embeddings, pooled_prompt_embeds,
           add_time_ids, w1, b1, wc, w2, b2, wt, wp, wid, bc,
           sigmas, timesteps):
    B, C, H, W = initial_latents_nchw.shape
    HW = H * W
    nsteps = timesteps.shape[0]
    hidden = w1.shape[-1]
    CP = -(-C // 8) * 8

    sig = sigmas.astype(jnp.float32)

    x0 = initial_latents_nchw.astype(jnp.float32).reshape(B, C, HW)
    x0 = jnp.pad(x0, ((0, 0), (0, CP - C), (0, 0))) * jnp.sqrt(sig[0] ** 2 + 1.0)

    temb = _timestep_embedding(timesteps, wt.shape[0])
    t_part = temb @ wt
    p_part = pooled_prompt_embeds.astype(jnp.float32) @ wp
    i_part = add_time_ids.astype(jnp.float32) @ wid
    cond = jax.nn.silu(t_part[:, None, :] + (p_part + i_part)[None, :, :]
                       + bc[None, None, :])
    ctx = jnp.mean(text_embeddings.astype(jnp.float32), axis=1) @ wc
    add = cond + ctx[None] + b1.reshape(1, 1, hidden)
    addv = jnp.transpose(add, (1, 0, 2))[:, :, :, None]
    addv = addv.astype(jnp.bfloat16)

    inv_scale = 1.0 / jnp.sqrt(sig[:-1] ** 2 + 1.0)
    dtv = sig[1:] - sig[:-1]
    w1p = jnp.pad(w1, ((0, 0), (0, CP - C), (0, 0)))
    w1_i2c = jnp.transpose(w1p, (2, 0, 1)).reshape(hidden, 9 * CP)
    w1_steps = (w1_i2c[None] * inv_scale[:, None, None]).astype(jnp.bfloat16)
    w2p = jnp.pad(w2, ((0, 0), (0, 0), (0, CP - C)))
    w2cat = jnp.transpose(w2p, (0, 2, 1)).reshape(9 * CP, hidden)
    w2_steps = (w2cat[None] * dtv[:, None, None]).astype(jnp.bfloat16)
    b2v = jnp.pad(b2.reshape(-1), (0, CP - C)).reshape(1, CP, 1)
    b2_steps = b2v * dtv[:, None, None]

    m = _boundary_masks(H, W)
    mask9 = jnp.broadcast_to(m[:, None, :], (9, CP, HW)).reshape(9 * CP, HW)

    NCHAIN = max(d for d in (8, 4, 2, 1) if B % d == 0)
    out = pl.pallas_call(
        _make_denoise_kernel(H, W, hidden, CP, nsteps, NCHAIN),
        out_shape=jax.ShapeDtypeStruct((B, CP, HW), jnp.float32),
        grid=(B // NCHAIN,),
        in_specs=[
            pl.BlockSpec((NCHAIN, CP, HW), lambda b: (b, 0, 0)),
            pl.BlockSpec((NCHAIN, nsteps, hidden, 1),
                         lambda b: (b, 0, 0, 0)),
            pl.BlockSpec((9 * CP, HW), lambda b: (0, 0)),
            pl.BlockSpec((nsteps, hidden, 9 * CP), lambda b: (0, 0, 0)),
            pl.BlockSpec((nsteps, 9 * CP, hidden), lambda b: (0, 0, 0)),
            pl.BlockSpec((nsteps, CP, 1), lambda b: (0, 0, 0)),
        ],
        out_specs=pl.BlockSpec((NCHAIN, CP, HW), lambda b: (b, 0, 0)),
        compiler_params=pltpu.CompilerParams(
            dimension_semantics=("parallel",)),
    )(x0, addv, mask9, w1_steps, w2_steps, b2_steps)

    return out[:, :C, :].reshape(B, C, H, W)
